# baseline (speedup 1.0000x reference)
import functools

import jax
import jax.numpy as jnp
from jax import lax
from jax.experimental import pallas as pl
from jax.experimental.pallas import tpu as pltpu
from jax.experimental.pallas import tpu_sc as plsc

TIME_DIM = 64
NUM_TIME_BINS = 1000
MAX_TIME_DIFF = 26000000.0
BATCH = 16384
SEQ = 200

NW = 32
ROWS_PER_W = BATCH // NW
SUPER = 8
NSUPER = ROWS_PER_W // SUPER
LANES = 16
SEQ_PAD = 208
NFULL = SEQ // LANES
TAIL = SEQ - NFULL * LANES
VW = TIME_DIM // LANES
TABLE_WORDS = (NUM_TIME_BINS + 1) * TIME_DIM


def _sc_lookup(cur_hbm, nbr_hbm, table_hbm, out_hbm,
               table_v, cur_v, nbr_v, offs_v, rows_v, in_sem, out_sem):
    wid = lax.axis_index("s") * 2 + lax.axis_index("c")
    wrow = wid * ROWS_PER_W

    def issue_in(s, q):
        base = wrow + jnp.minimum(s, NSUPER - 1) * SUPER
        pltpu.async_copy(cur_hbm.at[pl.ds(base, SUPER)], cur_v.at[q], in_sem)
        pltpu.async_copy(nbr_hbm.at[pl.ds(base, SUPER)], nbr_v.at[q], in_sem)

    def wait_in(q):
        pltpu.make_async_copy(cur_hbm.at[pl.ds(0, SUPER)], cur_v.at[q], in_sem).wait()
        pltpu.make_async_copy(nbr_hbm.at[pl.ds(0, SUPER)], nbr_v.at[q], in_sem).wait()

    def wait_out(po):
        pltpu.make_async_copy(rows_v.at[po], out_hbm.at[pl.ds(0, 1)], out_sem).wait()

    def discretize_super(q):
        def row_body(r, carry):
            @plsc.parallel_loop(0, SEQ_PAD // LANES)
            def disc_body(g):
                s = g * LANES
                c16 = cur_v[q, r, pl.ds(s, LANES)]
                n16 = nbr_v[q, r, pl.ds(s, LANES)]
                d = c16 - n16
                cl = jnp.minimum(jnp.maximum(d, 0.0), MAX_TIME_DIFF)
                b = ((cl / MAX_TIME_DIFF) * NUM_TIME_BINS).astype(jnp.int32)
                b = jnp.maximum(jnp.minimum(b, NUM_TIME_BINS), 0)
                offs_v[q * SUPER + r, pl.ds(s, LANES)] = b * TIME_DIM
            return carry
        lax.fori_loop(0, SUPER, row_body, 0)

    def gather_row(rr, po):
        @plsc.parallel_loop(0, NFULL // 2)
        def g_body(gp):
            g = gp * 2
            offv0 = offs_v[rr, pl.ds(g * LANES, LANES)]
            offv1 = offs_v[rr, pl.ds((g + 1) * LANES, LANES)]
            packed = offv0 | (offv1 << 16)
            for u in range(LANES):
                s2 = packed[u]
                off0 = s2 & 0xFFFF
                off1 = lax.shift_right_logical(s2, 16)
                lane = (u % 2) * TIME_DIM
                for gg, off in ((g, off0), (g + 1, off1)):
                    sh = (gg * LANES + u) // 2
                    for c in range(VW):
                        rows_v[po, 0, sh, pl.ds(lane + c * LANES, LANES)] = (
                            table_v[pl.ds(off + c * LANES, LANES)])
        offv = offs_v[rr, pl.ds(NFULL * LANES, LANES)]
        for u in range(TAIL):
            ss = NFULL * LANES + u
            off = offv[u]
            sh = ss // 2
            lane = (u % 2) * TIME_DIM
            for c in range(VW):
                rows_v[po, 0, sh, pl.ds(lane + c * LANES, LANES)] = (
                    table_v[pl.ds(off + c * LANES, LANES)])

    def issue_out(row, po):
        pltpu.async_copy(rows_v.at[po], out_hbm.at[pl.ds(wrow + row, 1)], out_sem)

    pltpu.sync_copy(table_hbm, table_v)

    issue_in(0, 0)
    for po in (0, 1, 2, 3):
        issue_out(po, po)

    def pair_body(k, carry):
        for q in (0, 1):
            wait_in(q)
            issue_in(2 * k + q + 1, 1 - q)
            discretize_super(q)

        def rp_body(rp, carry2):
            for po in (0, 1, 2, 3):
                rr = rp * 4 + po
                wait_out(po)
                gather_row(rr, po)
                issue_out(2 * SUPER * k + rr, po)
            return carry2
        lax.fori_loop(0, 2 * SUPER // 4, rp_body, 0)
        return carry
    lax.fori_loop(0, NSUPER // 2, pair_body, 0)

    wait_in(0)
    for po in (0, 1, 2, 3):
        wait_out(po)


def kernel(current_times, neighbor_times, lpe_weight):
    mesh = plsc.VectorSubcoreMesh(core_axis_name="c", subcore_axis_name="s")
    k = functools.partial(
        pl.kernel,
        out_type=jax.ShapeDtypeStruct((BATCH, SEQ // 2, 2 * TIME_DIM), jnp.float32),
        mesh=mesh,
        scratch_types=[
            pltpu.VMEM((TABLE_WORDS,), jnp.float32),
            pltpu.VMEM((2, SUPER, SEQ), jnp.float32),
            pltpu.VMEM((2, SUPER, SEQ), jnp.float32),
            pltpu.VMEM((2 * SUPER, SEQ_PAD), jnp.int32),
            pltpu.VMEM((4, 1, SEQ // 2, 2 * TIME_DIM), jnp.float32),
            pltpu.SemaphoreType.DMA,
            pltpu.SemaphoreType.DMA,
        ],
        compiler_params=pltpu.CompilerParams(use_tc_tiling_on_sc=True),
    )(_sc_lookup)
    out = k(current_times, neighbor_times, lpe_weight.reshape(TABLE_WORDS))
    return out.reshape(BATCH, SEQ, TIME_DIM)


# speedup vs baseline: 1.1664x; 1.1664x over previous
import functools

import jax
import jax.numpy as jnp
from jax import lax
from jax.experimental import pallas as pl
from jax.experimental.pallas import tpu as pltpu
from jax.experimental.pallas import tpu_sc as plsc

TIME_DIM = 64
NUM_TIME_BINS = 1000
MAX_TIME_DIFF = 26000000.0
BATCH = 16384
SEQ = 200

NW = 32
ROWS_PER_W = BATCH // NW
SUPER = 8
NSUPER = ROWS_PER_W // SUPER
LANES = 16
SEQ_PAD = 208
NFULL = SEQ // LANES
TAIL = SEQ - NFULL * LANES
VW = TIME_DIM // LANES
TABLE_WORDS = (NUM_TIME_BINS + 1) * TIME_DIM


def _sc_lookup(cur_hbm, nbr_hbm, table_hbm, out_hbm,
               table_v, cur_v, nbr_v, offs_v, rows_v, in_sem, out_sem):
    wid = lax.axis_index("s") * 2 + lax.axis_index("c")
    wrow = wid * ROWS_PER_W

    def issue_in(s, q):
        base = wrow + jnp.minimum(s, NSUPER - 1) * SUPER
        pltpu.async_copy(cur_hbm.at[pl.ds(base, SUPER)], cur_v.at[q], in_sem)
        pltpu.async_copy(nbr_hbm.at[pl.ds(base, SUPER)], nbr_v.at[q], in_sem)

    def wait_in(q):
        pltpu.make_async_copy(cur_hbm.at[pl.ds(0, SUPER)], cur_v.at[q], in_sem).wait()
        pltpu.make_async_copy(nbr_hbm.at[pl.ds(0, SUPER)], nbr_v.at[q], in_sem).wait()

    def wait_out(po):
        pltpu.make_async_copy(rows_v.at[po], out_hbm.at[pl.ds(0, 1)], out_sem).wait()

    def discretize(q, r):
        @plsc.parallel_loop(0, SEQ_PAD // LANES)
        def disc_body(g):
            s = g * LANES
            c16 = cur_v[q, r, pl.ds(s, LANES)]
            n16 = nbr_v[q, r, pl.ds(s, LANES)]
            d = c16 - n16
            cl = jnp.minimum(jnp.maximum(d, 0.0), MAX_TIME_DIFF)
            b = ((cl / MAX_TIME_DIFF) * NUM_TIME_BINS).astype(jnp.int32)
            b = jnp.maximum(jnp.minimum(b, NUM_TIME_BINS), 0)
            offs_v[q, r, pl.ds(s, LANES)] = b * TIME_DIM

    def gather_row(q, r, po):
        @plsc.parallel_loop(0, NFULL)
        def g_body(g):
            offv = offs_v[q, r, pl.ds(g * LANES, LANES)]
            for u in range(LANES):
                ss = g * LANES + u
                off = offv[u]
                sh = ss // 2
                lane = (u % 2) * TIME_DIM
                for c in range(VW):
                    rows_v[po, 0, sh, pl.ds(lane + c * LANES, LANES)] = (
                        table_v[pl.ds(off + c * LANES, LANES)])
        offv = offs_v[q, r, pl.ds(NFULL * LANES, LANES)]
        for u in range(TAIL):
            ss = NFULL * LANES + u
            off = offv[u]
            sh = ss // 2
            lane = (u % 2) * TIME_DIM
            for c in range(VW):
                rows_v[po, 0, sh, pl.ds(lane + c * LANES, LANES)] = (
                    table_v[pl.ds(off + c * LANES, LANES)])

    def issue_out(row, po):
        pltpu.async_copy(rows_v.at[po], out_hbm.at[pl.ds(wrow + row, 1)], out_sem)

    def do_super(s, q):
        wait_in(q)
        issue_in(s + 1, 1 - q)

        def rp_body(rp, carry):
            for po in (0, 1, 2, 3):
                r = rp * 4 + po
                discretize(q, r)
                wait_out(po)
                gather_row(q, r, po)
                issue_out(s * SUPER + r, po)
            return carry
        lax.fori_loop(0, SUPER // 4, rp_body, 0)

    pltpu.sync_copy(table_hbm, table_v)

    issue_in(0, 0)
    for po in (0, 1, 2, 3):
        issue_out(po, po)

    def pair_body(k, carry):
        do_super(2 * k, 0)
        do_super(2 * k + 1, 1)
        return carry
    lax.fori_loop(0, NSUPER // 2, pair_body, 0)

    wait_in(0)
    for po in (0, 1, 2, 3):
        wait_out(po)


def kernel(current_times, neighbor_times, lpe_weight):
    mesh = plsc.VectorSubcoreMesh(core_axis_name="c", subcore_axis_name="s")
    k = functools.partial(
        pl.kernel,
        out_type=jax.ShapeDtypeStruct((BATCH, SEQ // 2, 2 * TIME_DIM), jnp.float32),
        mesh=mesh,
        scratch_types=[
            pltpu.VMEM((TABLE_WORDS,), jnp.float32),
            pltpu.VMEM((2, SUPER, SEQ), jnp.float32),
            pltpu.VMEM((2, SUPER, SEQ), jnp.float32),
            pltpu.VMEM((2, SUPER, SEQ_PAD), jnp.int32),
            pltpu.VMEM((4, 1, SEQ // 2, 2 * TIME_DIM), jnp.float32),
            pltpu.SemaphoreType.DMA,
            pltpu.SemaphoreType.DMA,
        ],
        compiler_params=pltpu.CompilerParams(use_tc_tiling_on_sc=True),
    )(_sc_lookup)
    out = k(current_times, neighbor_times, lpe_weight.reshape(TABLE_WORDS))
    return out.reshape(BATCH, SEQ, TIME_DIM)


# speedup vs baseline: 1.2133x; 1.0402x over previous
import functools

import jax
import jax.numpy as jnp
from jax import lax
from jax.experimental import pallas as pl
from jax.experimental.pallas import tpu as pltpu
from jax.experimental.pallas import tpu_sc as plsc

TIME_DIM = 64
NUM_TIME_BINS = 1000
MAX_TIME_DIFF = 26000000.0
BATCH = 16384
SEQ = 200

NW = 32
ROWS_PER_W = BATCH // NW
SUPER = 8
NSUPER = ROWS_PER_W // SUPER
LANES = 16
SEQ_PAD = 208
NFULL = SEQ // LANES
TAIL = SEQ - NFULL * LANES
VW = TIME_DIM // LANES
TABLE_WORDS = (NUM_TIME_BINS + 1) * TIME_DIM


def _sc_lookup(cur_hbm, nbr_hbm, table_hbm, out_hbm,
               table_v, cur_v, nbr_v, offs_v, rows_v, in_sem, out_sem):
    wid = lax.axis_index("s") * 2 + lax.axis_index("c")
    wrow = wid * ROWS_PER_W

    def issue_in(s, q):
        base = wrow + jnp.minimum(s, NSUPER - 1) * SUPER
        pltpu.async_copy(cur_hbm.at[pl.ds(base, SUPER)], cur_v.at[q], in_sem)
        pltpu.async_copy(nbr_hbm.at[pl.ds(base, SUPER)], nbr_v.at[q], in_sem)

    def wait_in(q):
        pltpu.make_async_copy(cur_hbm.at[pl.ds(0, SUPER)], cur_v.at[q], in_sem).wait()
        pltpu.make_async_copy(nbr_hbm.at[pl.ds(0, SUPER)], nbr_v.at[q], in_sem).wait()

    def wait_out(po):
        pltpu.make_async_copy(rows_v.at[po], out_hbm.at[pl.ds(0, 1)], out_sem).wait()

    def discretize(q, r):
        @plsc.parallel_loop(0, SEQ_PAD // LANES)
        def disc_body(g):
            s = g * LANES
            c16 = cur_v[q, r, pl.ds(s, LANES)]
            n16 = nbr_v[q, r, pl.ds(s, LANES)]
            d = c16 - n16
            cl = jnp.minimum(jnp.maximum(d, 0.0), MAX_TIME_DIFF)
            b = ((cl / MAX_TIME_DIFF) * NUM_TIME_BINS).astype(jnp.int32)
            b = jnp.maximum(jnp.minimum(b, NUM_TIME_BINS), 0)
            offs_v[q, r, pl.ds(s, LANES)] = b * TIME_DIM

    def gather_row(q, r, po):
        @plsc.parallel_loop(0, NFULL)
        def g_body(g):
            offv = offs_v[q, r, pl.ds(g * LANES, LANES)]
            for u in range(LANES):
                ss = g * LANES + u
                off = offv[u]
                sh = ss // 2
                lane = (u % 2) * TIME_DIM
                for c in range(VW):
                    rows_v[po, 0, sh, pl.ds(lane + c * LANES, LANES)] = (
                        table_v[pl.ds(off + c * LANES, LANES)])
        offv = offs_v[q, r, pl.ds(NFULL * LANES, LANES)]
        for u in range(TAIL):
            ss = NFULL * LANES + u
            off = offv[u]
            sh = ss // 2
            lane = (u % 2) * TIME_DIM
            for c in range(VW):
                rows_v[po, 0, sh, pl.ds(lane + c * LANES, LANES)] = (
                    table_v[pl.ds(off + c * LANES, LANES)])

    def issue_out(row, po):
        pltpu.async_copy(rows_v.at[po], out_hbm.at[pl.ds(wrow + row, 1)], out_sem)

    def do_super(s, q):
        wait_in(q)
        issue_in(s + 1, 1 - q)

        def rp_body(rp, carry):
            for po in (0, 1):
                r = rp * 2 + po
                discretize(q, r)
                wait_out(po)
                gather_row(q, r, po)
                issue_out(s * SUPER + r, po)
            return carry
        lax.fori_loop(0, SUPER // 2, rp_body, 0)

    pltpu.sync_copy(table_hbm, table_v)

    issue_in(0, 0)
    issue_out(0, 0)
    issue_out(1, 1)

    def pair_body(k, carry):
        do_super(2 * k, 0)
        do_super(2 * k + 1, 1)
        return carry
    lax.fori_loop(0, NSUPER // 2, pair_body, 0)

    wait_in(0)
    wait_out(0)
    wait_out(1)


def kernel(current_times, neighbor_times, lpe_weight):
    mesh = plsc.VectorSubcoreMesh(core_axis_name="c", subcore_axis_name="s")
    k = functools.partial(
        pl.kernel,
        out_type=jax.ShapeDtypeStruct((BATCH, SEQ // 2, 2 * TIME_DIM), jnp.float32),
        mesh=mesh,
        scratch_types=[
            pltpu.VMEM((TABLE_WORDS,), jnp.float32),
            pltpu.VMEM((2, SUPER, SEQ), jnp.float32),
            pltpu.VMEM((2, SUPER, SEQ), jnp.float32),
            pltpu.VMEM((2, SUPER, SEQ_PAD), jnp.int32),
            pltpu.VMEM((2, 1, SEQ // 2, 2 * TIME_DIM), jnp.float32),
            pltpu.SemaphoreType.DMA,
            pltpu.SemaphoreType.DMA,
        ],
        compiler_params=pltpu.CompilerParams(use_tc_tiling_on_sc=True),
    )(_sc_lookup)
    out = k(current_times, neighbor_times, lpe_weight.reshape(TABLE_WORDS))
    return out.reshape(BATCH, SEQ, TIME_DIM)
